# Initial kernel scaffold; baseline (speedup 1.0000x reference)
import jax
import jax.numpy as jnp
from jax.experimental import pallas as pl


def kernel(values, col_idx, max_x, bias_x):
    raise NotImplementedError("write your pallas kernel here")


# speedup vs baseline: 1.3398x; 1.3398x over previous
import jax
import jax.numpy as jnp
from jax import lax
from jax.experimental import pallas as pl
from jax.experimental.pallas import tpu as pltpu
from jax.experimental.pallas import tpu_sc as plsc

NNZ_K = 1638400
N_COLS = 1000000
COLS_PAD = 1 << 20
WIN = 1 << 16
NWIN = COLS_PAD // WIN
LANES = 16
EPS_K = 1e-5

NSUB = 16
NCORE = 2
PART_A = NNZ_K // NSUB
CH_A = 10240
NCH_A = PART_A // CH_A
VREGS_A = CH_A // LANES
MR = WIN // NSUB

NW_B = NCORE * NSUB
EP_B = NNZ_K // NW_B
CH_B = 5120
NCH_B = EP_B // CH_B
ROWS_B = CH_B // 128


def _scatter_max_body(values_hbm, col_hbm, maxx_hbm, m_hbm, t_hbm,
                      table, cbuf, vbuf, mbuf, tbuf):
  cid = lax.axis_index("c")
  sid = lax.axis_index("s")

  def pass_body(p, _):
    win_id = cid * (NWIN // NCORE) + p
    wbase = win_id * WIN

    def zero_body(i, _):
      table[pl.ds(i * LANES, LANES)] = jnp.zeros((LANES,), jnp.float32)
      return 0
    lax.fori_loop(0, WIN // LANES, zero_body, 0)

    def chunk_body(ch, _):
      e0 = sid * PART_A + ch * CH_A
      pltpu.sync_copy(col_hbm.at[pl.ds(e0, CH_A)], cbuf)
      pltpu.sync_copy(values_hbm.at[pl.ds(e0, CH_A)], vbuf)

      def vec_body(k, _):
        c = cbuf[pl.ds(k * LANES, LANES)]
        v = vbuf[pl.ds(k * LANES, LANES)]
        inw = (c >> 16) == win_id
        lc = c & (WIN - 1)
        a = jnp.abs(v)
        g = plsc.load_gather(table, [lc], mask=inw)
        m = jnp.maximum(g, a)
        plsc.store_scatter(table, [lc], m, mask=inw)
        g2 = plsc.load_gather(table, [lc], mask=inw)
        fail = inw & (g2 < m)

        def wcond(st):
          return jnp.any(st[0])

        def wbody(st):
          f, _g = st
          plsc.store_scatter(table, [lc], m, mask=f)
          g3 = plsc.load_gather(table, [lc], mask=f)
          return (f & (g3 < m), g3)

        lax.while_loop(wcond, wbody, (fail, g2))
        return 0
      lax.fori_loop(0, VREGS_A, vec_body, 0)
      return 0
    lax.fori_loop(0, NCH_A, chunk_body, 0)

    tb = (cid * NSUB + sid) * WIN
    pltpu.sync_copy(table, t_hbm.at[pl.ds(tb, WIN)])
    plsc.subcore_barrier()

    mb = sid * MR
    pltpu.sync_copy(maxx_hbm.at[pl.ds(wbase + mb, MR)], mbuf)
    for j in range(NSUB):
      jb = (cid * NSUB + j) * WIN
      pltpu.sync_copy(t_hbm.at[pl.ds(jb + mb, MR)], tbuf)

      def merge_body(i, _):
        sl = pl.ds(i * LANES, LANES)
        mbuf[sl] = jnp.maximum(mbuf[sl], tbuf[sl])
        return 0
      lax.fori_loop(0, MR // LANES, merge_body, 0)
    pltpu.sync_copy(mbuf, m_hbm.at[pl.ds(wbase + mb, MR)])
    plsc.subcore_barrier()
    return 0
  lax.fori_loop(0, NWIN // NCORE, pass_body, 0)


def _normalize_body(values_hbm, col2_hbm, m_hbm, bias_hbm, out_hbm,
                    cb, vb, gb, bb, ob, sem):
  cid = lax.axis_index("c")
  sid = lax.axis_index("s")
  wid = sid * NCORE + cid

  def chunk_body(ch, _):
    e0 = wid * EP_B + ch * CH_B
    r0 = wid * (EP_B // 128) + ch * ROWS_B
    pltpu.sync_copy(col2_hbm.at[pl.ds(r0, ROWS_B)], cb)
    pltpu.sync_copy(values_hbm.at[pl.ds(e0, CH_B)], vb)

    descs = []
    for j in range(ROWS_B):
      descs.append(
          pltpu.async_copy(m_hbm.at[cb.at[j]],
                           gb.at[pl.ds(j * 128, 128)], sem))
      descs.append(
          pltpu.async_copy(bias_hbm.at[cb.at[j]],
                           bb.at[pl.ds(j * 128, 128)], sem))
    for d in descs:
      d.wait()

    def vec_body(k, _):
      sl = pl.ds(k * LANES, LANES)
      g = gb[sl]
      v = vb[sl]
      b = bb[sl]
      q = v / jnp.maximum(g, EPS_K)
      q = jnp.minimum(jnp.maximum(q, -1.0), 1.0)
      ob[sl] = q + b
      return 0
    lax.fori_loop(0, CH_B // LANES, vec_body, 0)

    pltpu.sync_copy(ob, out_hbm.at[pl.ds(e0, CH_B)])
    return 0
  lax.fori_loop(0, NCH_B, chunk_body, 0)


def _make_kernels():
  mesh = plsc.VectorSubcoreMesh(core_axis_name="c", subcore_axis_name="s")
  params = pltpu.CompilerParams(needs_layout_passes=False)
  kernel_a = pl.kernel(
      _scatter_max_body,
      out_type=[
          jax.ShapeDtypeStruct((COLS_PAD,), jnp.float32),
          jax.ShapeDtypeStruct((NCORE * NSUB * WIN,), jnp.float32),
      ],
      mesh=mesh,
      compiler_params=params,
      scratch_types=[
          pltpu.VMEM((WIN,), jnp.float32),
          pltpu.VMEM((CH_A,), jnp.int32),
          pltpu.VMEM((CH_A,), jnp.float32),
          pltpu.VMEM((MR,), jnp.float32),
          pltpu.VMEM((MR,), jnp.float32),
      ],
  )
  kernel_b = pl.kernel(
      _normalize_body,
      out_type=jax.ShapeDtypeStruct((NNZ_K,), jnp.float32),
      mesh=mesh,
      compiler_params=params,
      scratch_types=[
          pltpu.VMEM((ROWS_B, 128), jnp.int32),
          pltpu.VMEM((CH_B,), jnp.float32),
          pltpu.VMEM((CH_B,), jnp.float32),
          pltpu.VMEM((CH_B,), jnp.float32),
          pltpu.VMEM((CH_B,), jnp.float32),
          pltpu.SemaphoreType.DMA,
      ],
  )
  return kernel_a, kernel_b


_KERNEL_A, _KERNEL_B = _make_kernels()


@jax.jit
def kernel(values, col_idx, max_x, bias_x):
  col = col_idx.astype(jnp.int32)
  maxx_p = jnp.pad(max_x, (0, COLS_PAD - N_COLS))
  bias_p = jnp.pad(bias_x, (0, COLS_PAD - N_COLS))
  col2 = col.reshape(NNZ_K // 128, 128)
  m, _ = _KERNEL_A(values, col, maxx_p)
  out = _KERNEL_B(values, col2, m, bias_p)
  return out


# speedup vs baseline: 2.2994x; 1.7162x over previous
import jax
import jax.numpy as jnp
from jax import lax
from jax.experimental import pallas as pl
from jax.experimental.pallas import tpu as pltpu
from jax.experimental.pallas import tpu_sc as plsc

NNZ_K = 1638400
N_COLS = 1000000
COLS_PAD = 1 << 20
WIN = 1 << 16
NWIN = COLS_PAD // WIN
LANES = 16
EPS_K = 1e-5

NSUB = 16
NCORE = 2
PART_A = NNZ_K // NSUB
CH_A = 10240
NCH_A = PART_A // CH_A
VREGS_A = CH_A // LANES
UNROLL_A = 4
MR = WIN // NSUB

NW_B = NCORE * NSUB
EP_B = NNZ_K // NW_B
CH_B = 5120
NCH_B = EP_B // CH_B
ROWS_B = CH_B // 128


def _scatter_max_body(values_hbm, col_hbm, maxx_hbm, m_hbm, t_hbm,
                      table, cbuf, vbuf, mbuf, tbuf):
  cid = lax.axis_index("c")
  sid = lax.axis_index("s")

  def pass_body(p, _):
    win_id = cid * (NWIN // NCORE) + p
    wbase = win_id * WIN

    def zero_body(i, _):
      table[pl.ds(i * LANES, LANES)] = jnp.zeros((LANES,), jnp.float32)
      return 0
    lax.fori_loop(0, WIN // LANES, zero_body, 0, unroll=8)

    def chunk_body(ch, _):
      e0 = sid * PART_A + ch * CH_A
      pltpu.sync_copy(col_hbm.at[pl.ds(e0, CH_A)], cbuf)
      pltpu.sync_copy(values_hbm.at[pl.ds(e0, CH_A)], vbuf)

      def vec_body(k4, _):
        lcs, ms, inws = [], [], []
        for u in range(UNROLL_A):
          sl = pl.ds((k4 * UNROLL_A + u) * LANES, LANES)
          c = cbuf[sl]
          v = vbuf[sl]
          inw = (c >> 16) == win_id
          lc = c & (WIN - 1)
          a = jnp.abs(v)
          g = plsc.load_gather(table, [lc], mask=inw)
          m = jnp.maximum(g, a)
          plsc.store_scatter(table, [lc], m, mask=inw)
          lcs.append(lc)
          ms.append(m)
          inws.append(inw)
        fails = []
        for u in range(UNROLL_A):
          g2 = plsc.load_gather(table, [lcs[u]], mask=inws[u])
          fails.append(inws[u] & (g2 < ms[u]))
        anyf = fails[0]
        for u in range(1, UNROLL_A):
          anyf = anyf | fails[u]

        @pl.when(jnp.any(anyf))
        def _fix():
          for u in range(UNROLL_A):
            lc, m = lcs[u], ms[u]

            def wcond(st):
              return jnp.any(st)

            def wbody(f):
              plsc.store_scatter(table, [lc], m, mask=f)
              g3 = plsc.load_gather(table, [lc], mask=f)
              return f & (g3 < m)

            lax.while_loop(wcond, wbody, fails[u])
        return 0
      lax.fori_loop(0, VREGS_A // UNROLL_A, vec_body, 0)
      return 0
    lax.fori_loop(0, NCH_A, chunk_body, 0)

    tb = (cid * NSUB + sid) * WIN
    pltpu.sync_copy(table, t_hbm.at[pl.ds(tb, WIN)])
    plsc.subcore_barrier()

    mb = sid * MR
    pltpu.sync_copy(maxx_hbm.at[pl.ds(wbase + mb, MR)], mbuf)
    for j in range(NSUB):
      jb = (cid * NSUB + j) * WIN
      pltpu.sync_copy(t_hbm.at[pl.ds(jb + mb, MR)], tbuf)

      def merge_body(i, _):
        sl = pl.ds(i * LANES, LANES)
        mbuf[sl] = jnp.maximum(mbuf[sl], tbuf[sl])
        return 0
      lax.fori_loop(0, MR // LANES, merge_body, 0, unroll=4)
    pltpu.sync_copy(mbuf, m_hbm.at[pl.ds(wbase + mb, MR)])
    plsc.subcore_barrier()
    return 0
  lax.fori_loop(0, NWIN // NCORE, pass_body, 0)


def _normalize_body(values_hbm, col2_hbm, m_hbm, bias_hbm, out_hbm,
                    cb, vb, gb, bb, ob, sem):
  cid = lax.axis_index("c")
  sid = lax.axis_index("s")
  wid = sid * NCORE + cid

  def chunk_body(ch, _):
    e0 = wid * EP_B + ch * CH_B
    r0 = wid * (EP_B // 128) + ch * ROWS_B
    pltpu.sync_copy(col2_hbm.at[pl.ds(r0, ROWS_B)], cb)
    pltpu.sync_copy(values_hbm.at[pl.ds(e0, CH_B)], vb)

    descs = []
    for j in range(ROWS_B):
      descs.append(
          pltpu.async_copy(m_hbm.at[cb.at[j]],
                           gb.at[pl.ds(j * 128, 128)], sem))
      descs.append(
          pltpu.async_copy(bias_hbm.at[cb.at[j]],
                           bb.at[pl.ds(j * 128, 128)], sem))
    for d in descs:
      d.wait()

    def vec_body(k, _):
      sl = pl.ds(k * LANES, LANES)
      g = gb[sl]
      v = vb[sl]
      b = bb[sl]
      q = v / jnp.maximum(g, EPS_K)
      q = jnp.minimum(jnp.maximum(q, -1.0), 1.0)
      ob[sl] = q + b
      return 0
    lax.fori_loop(0, CH_B // LANES, vec_body, 0)

    pltpu.sync_copy(ob, out_hbm.at[pl.ds(e0, CH_B)])
    return 0
  lax.fori_loop(0, NCH_B, chunk_body, 0)


def _make_kernels():
  mesh = plsc.VectorSubcoreMesh(core_axis_name="c", subcore_axis_name="s")
  params = pltpu.CompilerParams(needs_layout_passes=False)
  kernel_a = pl.kernel(
      _scatter_max_body,
      out_type=[
          jax.ShapeDtypeStruct((COLS_PAD,), jnp.float32),
          jax.ShapeDtypeStruct((NCORE * NSUB * WIN,), jnp.float32),
      ],
      mesh=mesh,
      compiler_params=params,
      scratch_types=[
          pltpu.VMEM((WIN,), jnp.float32),
          pltpu.VMEM((CH_A,), jnp.int32),
          pltpu.VMEM((CH_A,), jnp.float32),
          pltpu.VMEM((MR,), jnp.float32),
          pltpu.VMEM((MR,), jnp.float32),
      ],
  )
  kernel_b = pl.kernel(
      _normalize_body,
      out_type=jax.ShapeDtypeStruct((NNZ_K,), jnp.float32),
      mesh=mesh,
      compiler_params=params,
      scratch_types=[
          pltpu.VMEM((ROWS_B, 128), jnp.int32),
          pltpu.VMEM((CH_B,), jnp.float32),
          pltpu.VMEM((CH_B,), jnp.float32),
          pltpu.VMEM((CH_B,), jnp.float32),
          pltpu.VMEM((CH_B,), jnp.float32),
          pltpu.SemaphoreType.DMA,
      ],
  )
  return kernel_a, kernel_b


_KERNEL_A, _KERNEL_B = _make_kernels()


@jax.jit
def kernel(values, col_idx, max_x, bias_x):
  col = col_idx.astype(jnp.int32)
  maxx_p = jnp.pad(max_x, (0, COLS_PAD - N_COLS))
  bias_p = jnp.pad(bias_x, (0, COLS_PAD - N_COLS))
  col2 = col.reshape(NNZ_K // 128, 128)
  m, _ = _KERNEL_A(values, col, maxx_p)
  out = _KERNEL_B(values, col2, m, bias_p)
  return out


# speedup vs baseline: 2.5812x; 1.1226x over previous
import jax
import jax.numpy as jnp
from jax import lax
from jax.experimental import pallas as pl
from jax.experimental.pallas import tpu as pltpu
from jax.experimental.pallas import tpu_sc as plsc

NNZ_K = 1638400
N_COLS = 1000000
COLS_PAD = 1 << 20
WIN = 1 << 16
NWIN = COLS_PAD // WIN
LANES = 16
EPS_K = 1e-5

NSUB = 16
NCORE = 2
PART_A = NNZ_K // NSUB
CH_A = 10240
NCH_A = PART_A // CH_A
VREGS_A = CH_A // LANES
UNROLL_A = 8
MR = WIN // NSUB

NW_B = NCORE * NSUB
EP_B = NNZ_K // NW_B
CH_B = 5120
NCH_B = EP_B // CH_B
ROWS_B = CH_B // 128


def _scatter_max_body(values_hbm, col_hbm, maxx_hbm, m_hbm, t_hbm,
                      table, cbuf, vbuf, mbuf, tbuf):
  cid = lax.axis_index("c")
  sid = lax.axis_index("s")

  def pass_body(p, _):
    win_id = cid * (NWIN // NCORE) + p
    wbase = win_id * WIN

    def zero_body(i, _):
      table[pl.ds(i * LANES, LANES)] = jnp.zeros((LANES,), jnp.float32)
      return 0
    lax.fori_loop(0, WIN // LANES, zero_body, 0, unroll=8)

    def chunk_body(ch, _):
      e0 = sid * PART_A + ch * CH_A
      pltpu.sync_copy(col_hbm.at[pl.ds(e0, CH_A)], cbuf)
      pltpu.sync_copy(values_hbm.at[pl.ds(e0, CH_A)], vbuf)

      def vec_body(k4, _):
        lcs, ms, inws = [], [], []
        for u in range(UNROLL_A):
          sl = pl.ds((k4 * UNROLL_A + u) * LANES, LANES)
          c = cbuf[sl]
          v = vbuf[sl]
          inw = (c >> 16) == win_id
          lc = c & (WIN - 1)
          a = jnp.abs(v)
          g = plsc.load_gather(table, [lc], mask=inw)
          m = jnp.maximum(g, a)
          plsc.store_scatter(table, [lc], m, mask=inw)
          lcs.append(lc)
          ms.append(m)
          inws.append(inw)
        fails = []
        for u in range(UNROLL_A):
          g2 = plsc.load_gather(table, [lcs[u]], mask=inws[u])
          fails.append(inws[u] & (g2 < ms[u]))
        anyf = fails[0]
        for u in range(1, UNROLL_A):
          anyf = anyf | fails[u]

        @pl.when(jnp.any(anyf))
        def _fix():
          for u in range(UNROLL_A):
            lc, m = lcs[u], ms[u]

            def wcond(st):
              return jnp.any(st)

            def wbody(f):
              plsc.store_scatter(table, [lc], m, mask=f)
              g3 = plsc.load_gather(table, [lc], mask=f)
              return f & (g3 < m)

            lax.while_loop(wcond, wbody, fails[u])
        return 0
      lax.fori_loop(0, VREGS_A // UNROLL_A, vec_body, 0)
      return 0
    lax.fori_loop(0, NCH_A, chunk_body, 0)

    tb = (cid * NSUB + sid) * WIN
    pltpu.sync_copy(table, t_hbm.at[pl.ds(tb, WIN)])
    plsc.subcore_barrier()

    mb = sid * MR
    pltpu.sync_copy(maxx_hbm.at[pl.ds(wbase + mb, MR)], mbuf)
    for j in range(NSUB):
      jb = (cid * NSUB + j) * WIN
      pltpu.sync_copy(t_hbm.at[pl.ds(jb + mb, MR)], tbuf)

      def merge_body(i, _):
        sl = pl.ds(i * LANES, LANES)
        mbuf[sl] = jnp.maximum(mbuf[sl], tbuf[sl])
        return 0
      lax.fori_loop(0, MR // LANES, merge_body, 0, unroll=4)
    pltpu.sync_copy(mbuf, m_hbm.at[pl.ds(wbase + mb, MR)])
    plsc.subcore_barrier()
    return 0
  lax.fori_loop(0, NWIN // NCORE, pass_body, 0)


def _normalize_body(values_hbm, col2_hbm, m_hbm, bias_hbm, out_hbm,
                    cb, vb, gb, bb, ob, sem):
  cid = lax.axis_index("c")
  sid = lax.axis_index("s")
  wid = sid * NCORE + cid

  def chunk_body(ch, _):
    e0 = wid * EP_B + ch * CH_B
    r0 = wid * (EP_B // 128) + ch * ROWS_B
    pltpu.sync_copy(col2_hbm.at[pl.ds(r0, ROWS_B)], cb)
    pltpu.sync_copy(values_hbm.at[pl.ds(e0, CH_B)], vb)

    descs = []
    for j in range(ROWS_B):
      descs.append(
          pltpu.async_copy(m_hbm.at[cb.at[j]],
                           gb.at[pl.ds(j * 128, 128)], sem))
      descs.append(
          pltpu.async_copy(bias_hbm.at[cb.at[j]],
                           bb.at[pl.ds(j * 128, 128)], sem))
    for d in descs:
      d.wait()

    def vec_body(k, _):
      sl = pl.ds(k * LANES, LANES)
      g = gb[sl]
      v = vb[sl]
      b = bb[sl]
      q = v / jnp.maximum(g, EPS_K)
      q = jnp.minimum(jnp.maximum(q, -1.0), 1.0)
      ob[sl] = q + b
      return 0
    lax.fori_loop(0, CH_B // LANES, vec_body, 0)

    pltpu.sync_copy(ob, out_hbm.at[pl.ds(e0, CH_B)])
    return 0
  lax.fori_loop(0, NCH_B, chunk_body, 0)


def _make_kernels():
  mesh = plsc.VectorSubcoreMesh(core_axis_name="c", subcore_axis_name="s")
  params = pltpu.CompilerParams(needs_layout_passes=False)
  kernel_a = pl.kernel(
      _scatter_max_body,
      out_type=[
          jax.ShapeDtypeStruct((COLS_PAD,), jnp.float32),
          jax.ShapeDtypeStruct((NCORE * NSUB * WIN,), jnp.float32),
      ],
      mesh=mesh,
      compiler_params=params,
      scratch_types=[
          pltpu.VMEM((WIN,), jnp.float32),
          pltpu.VMEM((CH_A,), jnp.int32),
          pltpu.VMEM((CH_A,), jnp.float32),
          pltpu.VMEM((MR,), jnp.float32),
          pltpu.VMEM((MR,), jnp.float32),
      ],
  )
  kernel_b = pl.kernel(
      _normalize_body,
      out_type=jax.ShapeDtypeStruct((NNZ_K,), jnp.float32),
      mesh=mesh,
      compiler_params=params,
      scratch_types=[
          pltpu.VMEM((ROWS_B, 128), jnp.int32),
          pltpu.VMEM((CH_B,), jnp.float32),
          pltpu.VMEM((CH_B,), jnp.float32),
          pltpu.VMEM((CH_B,), jnp.float32),
          pltpu.VMEM((CH_B,), jnp.float32),
          pltpu.SemaphoreType.DMA,
      ],
  )
  return kernel_a, kernel_b


_KERNEL_A, _KERNEL_B = _make_kernels()


@jax.jit
def kernel(values, col_idx, max_x, bias_x):
  col = col_idx.astype(jnp.int32)
  maxx_p = jnp.pad(max_x, (0, COLS_PAD - N_COLS))
  bias_p = jnp.pad(bias_x, (0, COLS_PAD - N_COLS))
  col2 = col.reshape(NNZ_K // 128, 128)
  m, _ = _KERNEL_A(values, col, maxx_p)
  out = _KERNEL_B(values, col2, m, bias_p)
  return out


# speedup vs baseline: 2.7664x; 1.0717x over previous
import jax
import jax.numpy as jnp
from jax import lax
from jax.experimental import pallas as pl
from jax.experimental.pallas import tpu as pltpu
from jax.experimental.pallas import tpu_sc as plsc

NNZ_K = 1638400
N_COLS = 1000000
COLS_PAD = 1 << 20
WIN = 1 << 16
NWIN = COLS_PAD // WIN
HALF_BIT = 1 << 19
LANES = 16
EPS_K = 1e-5

NSUB = 16
NCORE = 2
NW = NCORE * NSUB

SLICE0 = NNZ_K // NW
CH0 = 2048
NCH0 = SLICE0 // CH0
REG = SLICE0 + 64

CH_A = 2048
UNROLL_A = 8

EP_B = NNZ_K // NW
CH_B = 5120
NCH_B = EP_B // CH_B
ROWS_B = CH_B // 128


def _popcnt(mask):
  return plsc.all_reduce_population_count(mask)[0]


def _bin_body(values_hbm, col_hbm, c_scr, v_scr, cnt_hbm,
              inc, inv, oc0, ov0, oc1, ov1, cntv):
  cid = lax.axis_index("c")
  sid = lax.axis_index("s")
  wid = sid * NCORE + cid
  base0 = (wid * 2 + 0) * REG
  base1 = (wid * 2 + 1) * REG
  sent0 = jnp.zeros((LANES,), jnp.int32)
  sent1 = jnp.full((LANES,), HALF_BIT, jnp.int32)
  zf = jnp.zeros((LANES,), jnp.float32)

  def chunk_body(ch, hoffs):
    hoff0, hoff1 = hoffs
    e0 = wid * SLICE0 + ch * CH0
    pltpu.sync_copy(col_hbm.at[pl.ds(e0, CH0)], inc)
    pltpu.sync_copy(values_hbm.at[pl.ds(e0, CH0)], inv)

    def vec_body(k, offs):
      off0, off1 = offs
      sl = pl.ds(k * LANES, LANES)
      c = inc[sl]
      v = inv[sl]
      m1 = c >= HALF_BIT
      m0 = jnp.logical_not(m1)
      plsc.store_compressed(oc0.at[pl.ds(off0, LANES)], c, mask=m0)
      plsc.store_compressed(ov0.at[pl.ds(off0, LANES)], v, mask=m0)
      plsc.store_compressed(oc1.at[pl.ds(off1, LANES)], c, mask=m1)
      plsc.store_compressed(ov1.at[pl.ds(off1, LANES)], v, mask=m1)
      n0 = _popcnt(m0)
      return (off0 + n0, off1 + (LANES - n0))
    off0, off1 = lax.fori_loop(0, CH0 // LANES, vec_body, (0, 0), unroll=2)

    oc0[pl.ds(off0, LANES)] = sent0
    ov0[pl.ds(off0, LANES)] = zf
    oc1[pl.ds(off1, LANES)] = sent1
    ov1[pl.ds(off1, LANES)] = zf
    off0p = (off0 + LANES - 1) & ~(LANES - 1)
    off1p = (off1 + LANES - 1) & ~(LANES - 1)
    d0 = pl.multiple_of(base0 + hoff0, LANES)
    d1 = pl.multiple_of(base1 + hoff1, LANES)
    pltpu.sync_copy(oc0, c_scr.at[pl.ds(d0, CH0 + LANES)])
    pltpu.sync_copy(ov0, v_scr.at[pl.ds(d0, CH0 + LANES)])
    pltpu.sync_copy(oc1, c_scr.at[pl.ds(d1, CH0 + LANES)])
    pltpu.sync_copy(ov1, v_scr.at[pl.ds(d1, CH0 + LANES)])
    return (hoff0 + off0p, hoff1 + off1p)

  hoff0, hoff1 = lax.fori_loop(0, NCH0, chunk_body, (0, 0))
  cntv[pl.ds(0, LANES)] = jnp.full((LANES,), hoff0, jnp.int32)
  pltpu.sync_copy(cntv, cnt_hbm.at[0, wid])
  cntv[pl.ds(0, LANES)] = jnp.full((LANES,), hoff1, jnp.int32)
  pltpu.sync_copy(cntv, cnt_hbm.at[1, wid])


def _scatter_max_body(c_scr, v_scr, cnt_hbm, maxx_hbm, m_hbm, t_hbm,
                      table, cbuf, vbuf, mbuf, tbuf, cntv):
  cid = lax.axis_index("c")
  sid = lax.axis_index("s")
  h = cid
  p0 = 2 * sid
  p1 = 2 * sid + 1

  pltpu.sync_copy(cnt_hbm.at[h, p0], cntv)
  n0 = cntv[pl.ds(0, LANES)][0]
  pltpu.sync_copy(cnt_hbm.at[h, p1], cntv)
  n1 = cntv[pl.ds(0, LANES)][0]
  rb0 = (p0 * 2 + h) * REG
  rb1 = (p1 * 2 + h) * REG

  def pass_body(p, _):
    win_id = cid * (NWIN // NCORE) + p
    wbase = win_id * WIN

    def zero_body(i, _):
      table[pl.ds(i * LANES, LANES)] = jnp.zeros((LANES,), jnp.float32)
      return 0
    lax.fori_loop(0, WIN // LANES, zero_body, 0, unroll=8)

    def scan_region(rbase, n):
      nch = (n + CH_A - 1) >> 11

      def chunk_body(ch, _):
        e0 = pl.multiple_of(rbase + ch * CH_A, LANES)
        pltpu.sync_copy(c_scr.at[pl.ds(e0, CH_A)], cbuf)
        pltpu.sync_copy(v_scr.at[pl.ds(e0, CH_A)], vbuf)
        nv = jnp.minimum(CH_A, n - ch * CH_A) >> 4

        def vec_body(k4, _):
          lcs, ms, inws = [], [], []
          for u in range(UNROLL_A):
            sl = pl.ds((k4 * UNROLL_A + u) * LANES, LANES)
            c = cbuf[sl]
            v = vbuf[sl]
            inw = (c >> 16) == win_id
            lc = c & (WIN - 1)
            a = jnp.abs(v)
            g = plsc.load_gather(table, [lc], mask=inw)
            m = jnp.maximum(g, a)
            plsc.store_scatter(table, [lc], m, mask=inw)
            lcs.append(lc)
            ms.append(m)
            inws.append(inw)
          fails = []
          for u in range(UNROLL_A):
            g2 = plsc.load_gather(table, [lcs[u]], mask=inws[u])
            fails.append(inws[u] & (g2 < ms[u]))
          anyf = fails[0]
          for u in range(1, UNROLL_A):
            anyf = anyf | fails[u]

          @pl.when(jnp.any(anyf))
          def _fix():
            for u in range(UNROLL_A):
              lc, m = lcs[u], ms[u]

              def wcond(st):
                return jnp.any(st)

              def wbody(f):
                plsc.store_scatter(table, [lc], m, mask=f)
                g3 = plsc.load_gather(table, [lc], mask=f)
                return f & (g3 < m)

              lax.while_loop(wcond, wbody, fails[u])
          return 0
        lax.fori_loop(0, nv // UNROLL_A, vec_body, 0)

        def vec_tail(k, _):
          sl = pl.ds(k * LANES, LANES)
          c = cbuf[sl]
          v = vbuf[sl]
          inw = (c >> 16) == win_id
          lc = c & (WIN - 1)
          a = jnp.abs(v)
          g = plsc.load_gather(table, [lc], mask=inw)
          m = jnp.maximum(g, a)
          plsc.store_scatter(table, [lc], m, mask=inw)
          g2 = plsc.load_gather(table, [lc], mask=inw)
          fail = inw & (g2 < m)

          def wcond(st):
            return jnp.any(st)

          def wbody(f):
            plsc.store_scatter(table, [lc], m, mask=f)
            g3 = plsc.load_gather(table, [lc], mask=f)
            return f & (g3 < m)

          lax.while_loop(wcond, wbody, fail)
          return 0
        lax.fori_loop((nv // UNROLL_A) * UNROLL_A, nv, vec_tail, 0)
        return 0
      lax.fori_loop(0, nch, chunk_body, 0)

    scan_region(rb0, n0)
    scan_region(rb1, n1)

    tb = (cid * NSUB + sid) * WIN
    pltpu.sync_copy(table, t_hbm.at[pl.ds(tb, WIN)])
    plsc.subcore_barrier()

    mb = sid * MR
    pltpu.sync_copy(maxx_hbm.at[pl.ds(wbase + mb, MR)], mbuf)
    for j in range(NSUB):
      jb = (cid * NSUB + j) * WIN
      pltpu.sync_copy(t_hbm.at[pl.ds(jb + mb, MR)], tbuf)

      def merge_body(i, _):
        sl = pl.ds(i * LANES, LANES)
        mbuf[sl] = jnp.maximum(mbuf[sl], tbuf[sl])
        return 0
      lax.fori_loop(0, MR // LANES, merge_body, 0, unroll=4)
    pltpu.sync_copy(mbuf, m_hbm.at[pl.ds(wbase + mb, MR)])
    plsc.subcore_barrier()
    return 0
  lax.fori_loop(0, NWIN // NCORE, pass_body, 0)


MR = WIN // NSUB


def _normalize_body(values_hbm, col2_hbm, m_hbm, bias_hbm, out_hbm,
                    cb, vb, gb, bb, ob, sem):
  cid = lax.axis_index("c")
  sid = lax.axis_index("s")
  wid = sid * NCORE + cid

  def chunk_body(ch, _):
    e0 = wid * EP_B + ch * CH_B
    r0 = wid * (EP_B // 128) + ch * ROWS_B
    pltpu.sync_copy(col2_hbm.at[pl.ds(r0, ROWS_B)], cb)
    pltpu.sync_copy(values_hbm.at[pl.ds(e0, CH_B)], vb)

    descs = []
    for j in range(ROWS_B):
      descs.append(
          pltpu.async_copy(m_hbm.at[cb.at[j]],
                           gb.at[pl.ds(j * 128, 128)], sem))
      descs.append(
          pltpu.async_copy(bias_hbm.at[cb.at[j]],
                           bb.at[pl.ds(j * 128, 128)], sem))
    for d in descs:
      d.wait()

    def vec_body(k, _):
      sl = pl.ds(k * LANES, LANES)
      g = gb[sl]
      v = vb[sl]
      b = bb[sl]
      q = v / jnp.maximum(g, EPS_K)
      q = jnp.minimum(jnp.maximum(q, -1.0), 1.0)
      ob[sl] = q + b
      return 0
    lax.fori_loop(0, CH_B // LANES, vec_body, 0, unroll=4)

    pltpu.sync_copy(ob, out_hbm.at[pl.ds(e0, CH_B)])
    return 0
  lax.fori_loop(0, NCH_B, chunk_body, 0)


def _make_kernels():
  mesh = plsc.VectorSubcoreMesh(core_axis_name="c", subcore_axis_name="s")
  params = pltpu.CompilerParams(needs_layout_passes=False)
  kernel_a0 = pl.kernel(
      _bin_body,
      out_type=[
          jax.ShapeDtypeStruct((NW * 2 * REG,), jnp.int32),
          jax.ShapeDtypeStruct((NW * 2 * REG,), jnp.float32),
          jax.ShapeDtypeStruct((2, NW, LANES), jnp.int32),
      ],
      mesh=mesh,
      compiler_params=params,
      scratch_types=[
          pltpu.VMEM((CH0,), jnp.int32),
          pltpu.VMEM((CH0,), jnp.float32),
          pltpu.VMEM((CH0 + LANES,), jnp.int32),
          pltpu.VMEM((CH0 + LANES,), jnp.float32),
          pltpu.VMEM((CH0 + LANES,), jnp.int32),
          pltpu.VMEM((CH0 + LANES,), jnp.float32),
          pltpu.VMEM((LANES,), jnp.int32),
      ],
  )
  kernel_a1 = pl.kernel(
      _scatter_max_body,
      out_type=[
          jax.ShapeDtypeStruct((COLS_PAD,), jnp.float32),
          jax.ShapeDtypeStruct((NCORE * NSUB * WIN,), jnp.float32),
      ],
      mesh=mesh,
      compiler_params=params,
      scratch_types=[
          pltpu.VMEM((WIN,), jnp.float32),
          pltpu.VMEM((CH_A,), jnp.int32),
          pltpu.VMEM((CH_A,), jnp.float32),
          pltpu.VMEM((MR,), jnp.float32),
          pltpu.VMEM((MR,), jnp.float32),
          pltpu.VMEM((LANES,), jnp.int32),
      ],
  )
  kernel_b = pl.kernel(
      _normalize_body,
      out_type=jax.ShapeDtypeStruct((NNZ_K,), jnp.float32),
      mesh=mesh,
      compiler_params=params,
      scratch_types=[
          pltpu.VMEM((ROWS_B, 128), jnp.int32),
          pltpu.VMEM((CH_B,), jnp.float32),
          pltpu.VMEM((CH_B,), jnp.float32),
          pltpu.VMEM((CH_B,), jnp.float32),
          pltpu.VMEM((CH_B,), jnp.float32),
          pltpu.SemaphoreType.DMA,
      ],
  )
  return kernel_a0, kernel_a1, kernel_b


_KERNEL_A0, _KERNEL_A1, _KERNEL_B = _make_kernels()


@jax.jit
def kernel(values, col_idx, max_x, bias_x):
  col = col_idx.astype(jnp.int32)
  maxx_p = jnp.pad(max_x, (0, COLS_PAD - N_COLS))
  bias_p = jnp.pad(bias_x, (0, COLS_PAD - N_COLS))
  col2 = col.reshape(NNZ_K // 128, 128)
  c_scr, v_scr, cnts = _KERNEL_A0(values, col)
  m, _ = _KERNEL_A1(c_scr, v_scr, cnts, maxx_p)
  out = _KERNEL_B(values, col2, m, bias_p)
  return out


# speedup vs baseline: 3.3637x; 1.2159x over previous
import jax
import jax.numpy as jnp
from jax import lax
from jax.experimental import pallas as pl
from jax.experimental.pallas import tpu as pltpu
from jax.experimental.pallas import tpu_sc as plsc

NNZ_K = 1638400
N_COLS = 1000000
COLS_PAD = 1 << 20
WIN = 1 << 16
NWIN = COLS_PAD // WIN
HALF_BIT = 1 << 19
LANES = 16
EPS_K = 1e-5

NSUB = 16
NCORE = 2
NW = NCORE * NSUB

SLICE0 = NNZ_K // NW
CH0 = 2048
NCH0 = SLICE0 // CH0
REG = SLICE0 + 64

CH_A = 2048
UNROLL_A = 8

EP_B = NNZ_K // NW
CH_B = 5120
NCH_B = EP_B // CH_B
ROWS_B = CH_B // 128


def _popcnt(mask):
  return plsc.all_reduce_population_count(mask)[0]


def _bin_body(values_hbm, col_hbm, c_scr, v_scr, cnt_hbm,
              inc, inv, oc0, ov0, oc1, ov1, cntv):
  cid = lax.axis_index("c")
  sid = lax.axis_index("s")
  wid = sid * NCORE + cid
  base0 = (wid * 2 + 0) * REG
  base1 = (wid * 2 + 1) * REG
  sent0 = jnp.zeros((LANES,), jnp.int32)
  sent1 = jnp.full((LANES,), HALF_BIT, jnp.int32)
  zf = jnp.zeros((LANES,), jnp.float32)

  def chunk_body(ch, hoffs):
    hoff0, hoff1 = hoffs
    e0 = wid * SLICE0 + ch * CH0
    pltpu.sync_copy(col_hbm.at[pl.ds(e0, CH0)], inc)
    pltpu.sync_copy(values_hbm.at[pl.ds(e0, CH0)], inv)

    def vec_body(k, offs):
      off0, off1 = offs
      sl = pl.ds(k * LANES, LANES)
      c = inc[sl]
      v = inv[sl]
      m1 = c >= HALF_BIT
      m0 = jnp.logical_not(m1)
      plsc.store_compressed(oc0.at[pl.ds(off0, LANES)], c, mask=m0)
      plsc.store_compressed(ov0.at[pl.ds(off0, LANES)], v, mask=m0)
      plsc.store_compressed(oc1.at[pl.ds(off1, LANES)], c, mask=m1)
      plsc.store_compressed(ov1.at[pl.ds(off1, LANES)], v, mask=m1)
      n0 = _popcnt(m0)
      return (off0 + n0, off1 + (LANES - n0))
    off0, off1 = lax.fori_loop(0, CH0 // LANES, vec_body, (0, 0), unroll=2)

    oc0[pl.ds(off0, LANES)] = sent0
    ov0[pl.ds(off0, LANES)] = zf
    oc1[pl.ds(off1, LANES)] = sent1
    ov1[pl.ds(off1, LANES)] = zf
    off0p = (off0 + LANES - 1) & ~(LANES - 1)
    off1p = (off1 + LANES - 1) & ~(LANES - 1)
    d0 = pl.multiple_of(base0 + hoff0, LANES)
    d1 = pl.multiple_of(base1 + hoff1, LANES)
    pltpu.sync_copy(oc0, c_scr.at[pl.ds(d0, CH0 + LANES)])
    pltpu.sync_copy(ov0, v_scr.at[pl.ds(d0, CH0 + LANES)])
    pltpu.sync_copy(oc1, c_scr.at[pl.ds(d1, CH0 + LANES)])
    pltpu.sync_copy(ov1, v_scr.at[pl.ds(d1, CH0 + LANES)])
    return (hoff0 + off0p, hoff1 + off1p)

  hoff0, hoff1 = lax.fori_loop(0, NCH0, chunk_body, (0, 0))
  cntv[pl.ds(0, LANES)] = jnp.full((LANES,), hoff0, jnp.int32)
  pltpu.sync_copy(cntv, cnt_hbm.at[0, wid])
  cntv[pl.ds(0, LANES)] = jnp.full((LANES,), hoff1, jnp.int32)
  pltpu.sync_copy(cntv, cnt_hbm.at[1, wid])


def _scatter_max_body(c_scr, v_scr, cnt_hbm, maxx_hbm, m_hbm, t_hbm,
                      table, cbuf, vbuf, mbuf, tbuf, cntv):
  cid = lax.axis_index("c")
  sid = lax.axis_index("s")
  h = cid
  p0 = 2 * sid
  p1 = 2 * sid + 1

  pltpu.sync_copy(cnt_hbm.at[h, p0], cntv)
  n0 = cntv[pl.ds(0, LANES)][0]
  pltpu.sync_copy(cnt_hbm.at[h, p1], cntv)
  n1 = cntv[pl.ds(0, LANES)][0]
  rb0 = (p0 * 2 + h) * REG
  rb1 = (p1 * 2 + h) * REG

  def pass_body(p, _):
    win_id = cid * (NWIN // NCORE) + p
    wbase = win_id * WIN

    def zero_body(i, _):
      table[pl.ds(i * LANES, LANES)] = jnp.zeros((LANES,), jnp.float32)
      return 0
    lax.fori_loop(0, WIN // LANES, zero_body, 0, unroll=8)

    def scan_region(rbase, n):
      nch = (n + CH_A - 1) >> 11

      def chunk_body(ch, _):
        e0 = pl.multiple_of(rbase + ch * CH_A, LANES)
        pltpu.sync_copy(c_scr.at[pl.ds(e0, CH_A)], cbuf)
        pltpu.sync_copy(v_scr.at[pl.ds(e0, CH_A)], vbuf)
        nv = jnp.minimum(CH_A, n - ch * CH_A) >> 4

        def vec_body(k4, _):
          lcs, inws, avs = [], [], []
          for u in range(UNROLL_A):
            sl = pl.ds((k4 * UNROLL_A + u) * LANES, LANES)
            c = cbuf[sl]
            v = vbuf[sl]
            inws.append((c >> 16) == win_id)
            lcs.append(c & (WIN - 1))
            avs.append(jnp.abs(v))
          gs = [plsc.load_gather(table, [lcs[u]], mask=inws[u])
                for u in range(UNROLL_A)]
          ms = [jnp.maximum(gs[u], avs[u]) for u in range(UNROLL_A)]
          for u in range(UNROLL_A):
            plsc.store_scatter(table, [lcs[u]], ms[u], mask=inws[u])
          fails = []
          for u in range(UNROLL_A):
            g2 = plsc.load_gather(table, [lcs[u]], mask=inws[u])
            fails.append(inws[u] & (g2 < ms[u]))
          anyf = fails[0]
          for u in range(1, UNROLL_A):
            anyf = anyf | fails[u]

          @pl.when(jnp.any(anyf))
          def _fix():
            for u in range(UNROLL_A):
              lc, m = lcs[u], ms[u]

              def wcond(st):
                return jnp.any(st)

              def wbody(f):
                g3 = plsc.load_gather(table, [lc], mask=f)
                plsc.store_scatter(table, [lc], jnp.maximum(g3, m), mask=f)
                g4 = plsc.load_gather(table, [lc], mask=f)
                return f & (g4 < m)

              lax.while_loop(wcond, wbody, fails[u])
          return 0
        lax.fori_loop(0, nv // UNROLL_A, vec_body, 0)

        def vec_tail(k, _):
          sl = pl.ds(k * LANES, LANES)
          c = cbuf[sl]
          v = vbuf[sl]
          inw = (c >> 16) == win_id
          lc = c & (WIN - 1)
          a = jnp.abs(v)
          g = plsc.load_gather(table, [lc], mask=inw)
          m = jnp.maximum(g, a)
          plsc.store_scatter(table, [lc], m, mask=inw)
          g2 = plsc.load_gather(table, [lc], mask=inw)
          fail = inw & (g2 < m)

          def wcond(st):
            return jnp.any(st)

          def wbody(f):
            plsc.store_scatter(table, [lc], m, mask=f)
            g3 = plsc.load_gather(table, [lc], mask=f)
            return f & (g3 < m)

          lax.while_loop(wcond, wbody, fail)
          return 0
        lax.fori_loop((nv // UNROLL_A) * UNROLL_A, nv, vec_tail, 0)
        return 0
      lax.fori_loop(0, nch, chunk_body, 0)

    scan_region(rb0, n0)
    scan_region(rb1, n1)

    tb = (cid * NSUB + sid) * WIN
    pltpu.sync_copy(table, t_hbm.at[pl.ds(tb, WIN)])
    plsc.subcore_barrier()

    mb = sid * MR
    pltpu.sync_copy(maxx_hbm.at[pl.ds(wbase + mb, MR)], mbuf)
    for j in range(NSUB):
      jb = (cid * NSUB + j) * WIN
      pltpu.sync_copy(t_hbm.at[pl.ds(jb + mb, MR)], tbuf)

      def merge_body(i, _):
        sl = pl.ds(i * LANES, LANES)
        mbuf[sl] = jnp.maximum(mbuf[sl], tbuf[sl])
        return 0
      lax.fori_loop(0, MR // LANES, merge_body, 0, unroll=4)
    pltpu.sync_copy(mbuf, m_hbm.at[pl.ds(wbase + mb, MR)])
    plsc.subcore_barrier()
    return 0
  lax.fori_loop(0, NWIN // NCORE, pass_body, 0)


MR = WIN // NSUB


def _normalize_body(values_hbm, col2_hbm, m_hbm, bias_hbm, out_hbm,
                    cb, vb, gb, bb, ob, sem):
  cid = lax.axis_index("c")
  sid = lax.axis_index("s")
  wid = sid * NCORE + cid

  def chunk_body(ch, _):
    e0 = wid * EP_B + ch * CH_B
    r0 = wid * (EP_B // 128) + ch * ROWS_B
    pltpu.sync_copy(col2_hbm.at[pl.ds(r0, ROWS_B)], cb)
    pltpu.sync_copy(values_hbm.at[pl.ds(e0, CH_B)], vb)

    descs = []
    for j in range(ROWS_B):
      descs.append(
          pltpu.async_copy(m_hbm.at[cb.at[j]],
                           gb.at[pl.ds(j * 128, 128)], sem))
      descs.append(
          pltpu.async_copy(bias_hbm.at[cb.at[j]],
                           bb.at[pl.ds(j * 128, 128)], sem))
    for d in descs:
      d.wait()

    def vec_body(k, _):
      sl = pl.ds(k * LANES, LANES)
      g = gb[sl]
      v = vb[sl]
      b = bb[sl]
      q = v / jnp.maximum(g, EPS_K)
      q = jnp.minimum(jnp.maximum(q, -1.0), 1.0)
      ob[sl] = q + b
      return 0
    lax.fori_loop(0, CH_B // LANES, vec_body, 0, unroll=4)

    pltpu.sync_copy(ob, out_hbm.at[pl.ds(e0, CH_B)])
    return 0
  lax.fori_loop(0, NCH_B, chunk_body, 0)


def _make_kernels():
  mesh = plsc.VectorSubcoreMesh(core_axis_name="c", subcore_axis_name="s")
  params = pltpu.CompilerParams(needs_layout_passes=False)
  kernel_a0 = pl.kernel(
      _bin_body,
      out_type=[
          jax.ShapeDtypeStruct((NW * 2 * REG,), jnp.int32),
          jax.ShapeDtypeStruct((NW * 2 * REG,), jnp.float32),
          jax.ShapeDtypeStruct((2, NW, LANES), jnp.int32),
      ],
      mesh=mesh,
      compiler_params=params,
      scratch_types=[
          pltpu.VMEM((CH0,), jnp.int32),
          pltpu.VMEM((CH0,), jnp.float32),
          pltpu.VMEM((CH0 + LANES,), jnp.int32),
          pltpu.VMEM((CH0 + LANES,), jnp.float32),
          pltpu.VMEM((CH0 + LANES,), jnp.int32),
          pltpu.VMEM((CH0 + LANES,), jnp.float32),
          pltpu.VMEM((LANES,), jnp.int32),
      ],
  )
  kernel_a1 = pl.kernel(
      _scatter_max_body,
      out_type=[
          jax.ShapeDtypeStruct((COLS_PAD,), jnp.float32),
          jax.ShapeDtypeStruct((NCORE * NSUB * WIN,), jnp.float32),
      ],
      mesh=mesh,
      compiler_params=params,
      scratch_types=[
          pltpu.VMEM((WIN,), jnp.float32),
          pltpu.VMEM((CH_A,), jnp.int32),
          pltpu.VMEM((CH_A,), jnp.float32),
          pltpu.VMEM((MR,), jnp.float32),
          pltpu.VMEM((MR,), jnp.float32),
          pltpu.VMEM((LANES,), jnp.int32),
      ],
  )
  kernel_b = pl.kernel(
      _normalize_body,
      out_type=jax.ShapeDtypeStruct((NNZ_K,), jnp.float32),
      mesh=mesh,
      compiler_params=params,
      scratch_types=[
          pltpu.VMEM((ROWS_B, 128), jnp.int32),
          pltpu.VMEM((CH_B,), jnp.float32),
          pltpu.VMEM((CH_B,), jnp.float32),
          pltpu.VMEM((CH_B,), jnp.float32),
          pltpu.VMEM((CH_B,), jnp.float32),
          pltpu.SemaphoreType.DMA,
      ],
  )
  return kernel_a0, kernel_a1, kernel_b


_KERNEL_A0, _KERNEL_A1, _KERNEL_B = _make_kernels()


@jax.jit
def kernel(values, col_idx, max_x, bias_x):
  col = col_idx.astype(jnp.int32)
  maxx_p = jnp.pad(max_x, (0, COLS_PAD - N_COLS))
  bias_p = jnp.pad(bias_x, (0, COLS_PAD - N_COLS))
  col2 = col.reshape(NNZ_K // 128, 128)
  c_scr, v_scr, cnts = _KERNEL_A0(values, col)
  m, _ = _KERNEL_A1(c_scr, v_scr, cnts, maxx_p)
  out = _KERNEL_B(values, col2, m, bias_p)
  return out


# speedup vs baseline: 4.9863x; 1.4824x over previous
import jax
import jax.numpy as jnp
from jax import lax
from jax.experimental import pallas as pl
from jax.experimental.pallas import tpu as pltpu
from jax.experimental.pallas import tpu_sc as plsc

NNZ_K = 1638400
N_COLS = 1000000
COLS_PAD = 1 << 20
WIN = 1 << 16
NWIN = COLS_PAD // WIN
HALF_BIT = 1 << 19
LANES = 16
EPS_K = 1e-5

NSUB = 16
NCORE = 2
NW = NCORE * NSUB

SLICE0 = NNZ_K // NW
CH0 = 2048
NCH0 = SLICE0 // CH0
REG = SLICE0 + 64

CH_A = 8192
UNROLL_A = 8
MERGE_B = 8

EP_B = NNZ_K // NW
CH_B = 5120
NCH_B = EP_B // CH_B
ROWS_B = CH_B // 128


def _popcnt(mask):
  return plsc.all_reduce_population_count(mask)[0]


def _bin_body(values_hbm, col_hbm, c_scr, v_scr, cnt_hbm,
              inc, inv, oc0, ov0, oc1, ov1, cntv):
  cid = lax.axis_index("c")
  sid = lax.axis_index("s")
  wid = sid * NCORE + cid
  base0 = (wid * 2 + 0) * REG
  base1 = (wid * 2 + 1) * REG
  sent0 = jnp.zeros((LANES,), jnp.int32)
  sent1 = jnp.full((LANES,), HALF_BIT, jnp.int32)
  zf = jnp.zeros((LANES,), jnp.float32)

  def chunk_body(ch, hoffs):
    hoff0, hoff1 = hoffs
    e0 = wid * SLICE0 + ch * CH0
    pltpu.sync_copy(col_hbm.at[pl.ds(e0, CH0)], inc)
    pltpu.sync_copy(values_hbm.at[pl.ds(e0, CH0)], inv)

    def vec_body(k, offs):
      off0, off1 = offs
      sl = pl.ds(k * LANES, LANES)
      c = inc[sl]
      v = inv[sl]
      m1 = c >= HALF_BIT
      m0 = jnp.logical_not(m1)
      plsc.store_compressed(oc0.at[pl.ds(off0, LANES)], c, mask=m0)
      plsc.store_compressed(ov0.at[pl.ds(off0, LANES)], v, mask=m0)
      plsc.store_compressed(oc1.at[pl.ds(off1, LANES)], c, mask=m1)
      plsc.store_compressed(ov1.at[pl.ds(off1, LANES)], v, mask=m1)
      n0 = _popcnt(m0)
      return (off0 + n0, off1 + (LANES - n0))
    off0, off1 = lax.fori_loop(0, CH0 // LANES, vec_body, (0, 0), unroll=2)

    oc0[pl.ds(off0, LANES)] = sent0
    ov0[pl.ds(off0, LANES)] = zf
    oc1[pl.ds(off1, LANES)] = sent1
    ov1[pl.ds(off1, LANES)] = zf
    off0p = (off0 + LANES - 1) & ~(LANES - 1)
    off1p = (off1 + LANES - 1) & ~(LANES - 1)
    d0 = pl.multiple_of(base0 + hoff0, LANES)
    d1 = pl.multiple_of(base1 + hoff1, LANES)
    pltpu.sync_copy(oc0, c_scr.at[pl.ds(d0, CH0 + LANES)])
    pltpu.sync_copy(ov0, v_scr.at[pl.ds(d0, CH0 + LANES)])
    pltpu.sync_copy(oc1, c_scr.at[pl.ds(d1, CH0 + LANES)])
    pltpu.sync_copy(ov1, v_scr.at[pl.ds(d1, CH0 + LANES)])
    return (hoff0 + off0p, hoff1 + off1p)

  hoff0, hoff1 = lax.fori_loop(0, NCH0, chunk_body, (0, 0))
  cntv[pl.ds(0, LANES)] = jnp.full((LANES,), hoff0, jnp.int32)
  pltpu.sync_copy(cntv, cnt_hbm.at[0, wid])
  cntv[pl.ds(0, LANES)] = jnp.full((LANES,), hoff1, jnp.int32)
  pltpu.sync_copy(cntv, cnt_hbm.at[1, wid])


def _scatter_max_body(c_scr, v_scr, cnt_hbm, maxx_hbm, m_hbm, t_hbm,
                      table, cbuf, vbuf, mbuf, tbuf, cntv, sem_a):
  cid = lax.axis_index("c")
  sid = lax.axis_index("s")
  h = cid
  p0 = 2 * sid
  p1 = 2 * sid + 1

  pltpu.sync_copy(cnt_hbm.at[h, p0], cntv)
  n0 = cntv[pl.ds(0, LANES)][0]
  pltpu.sync_copy(cnt_hbm.at[h, p1], cntv)
  n1 = cntv[pl.ds(0, LANES)][0]
  rb0 = (p0 * 2 + h) * REG
  rb1 = (p1 * 2 + h) * REG

  def pass_body(p, _):
    win_id = cid * (NWIN // NCORE) + p
    wbase = win_id * WIN

    def zero_body(i, _):
      table[pl.ds(i * LANES, LANES)] = jnp.zeros((LANES,), jnp.float32)
      return 0
    lax.fori_loop(0, WIN // LANES, zero_body, 0, unroll=8)

    def scan_region(rbase, n):
      nch = (n + CH_A - 1) >> 13

      def chunk_body(ch, _):
        e0 = pl.multiple_of(rbase + ch * CH_A, LANES)
        pltpu.sync_copy(c_scr.at[pl.ds(e0, CH_A)], cbuf)
        pltpu.sync_copy(v_scr.at[pl.ds(e0, CH_A)], vbuf)
        nv = jnp.minimum(CH_A, n - ch * CH_A) >> 4


        def vec_body(k4, _):
          lcs, inws, avs = [], [], []
          for u in range(UNROLL_A):
            sl = pl.ds((k4 * UNROLL_A + u) * LANES, LANES)
            c = cbuf[sl]
            v = vbuf[sl]
            inws.append((c >> 16) == win_id)
            lcs.append(c & (WIN - 1))
            avs.append(jnp.abs(v))
          gs = [plsc.load_gather(table, [lcs[u]], mask=inws[u])
                for u in range(UNROLL_A)]
          ms = [jnp.maximum(gs[u], avs[u]) for u in range(UNROLL_A)]
          for u in range(UNROLL_A):
            plsc.store_scatter(table, [lcs[u]], ms[u], mask=inws[u])
          fails = []
          for u in range(UNROLL_A):
            g2 = plsc.load_gather(table, [lcs[u]], mask=inws[u])
            fails.append(inws[u] & (g2 < ms[u]))
          anyf = fails[0]
          for u in range(1, UNROLL_A):
            anyf = anyf | fails[u]

          @pl.when(jnp.any(anyf))
          def _fix():
            for u in range(UNROLL_A):
              lc, m = lcs[u], ms[u]

              def wcond(st):
                return jnp.any(st)

              def wbody(f):
                g3 = plsc.load_gather(table, [lc], mask=f)
                plsc.store_scatter(table, [lc], jnp.maximum(g3, m), mask=f)
                g4 = plsc.load_gather(table, [lc], mask=f)
                return f & (g4 < m)

              lax.while_loop(wcond, wbody, fails[u])
          return 0
        lax.fori_loop(0, nv // UNROLL_A, vec_body, 0)

        def vec_tail(k, _):
          sl = pl.ds(k * LANES, LANES)
          c = cbuf[sl]
          v = vbuf[sl]
          inw = (c >> 16) == win_id
          lc = c & (WIN - 1)
          a = jnp.abs(v)
          g = plsc.load_gather(table, [lc], mask=inw)
          m = jnp.maximum(g, a)
          plsc.store_scatter(table, [lc], m, mask=inw)
          g2 = plsc.load_gather(table, [lc], mask=inw)
          fail = inw & (g2 < m)

          def wcond(st):
            return jnp.any(st)

          def wbody(f):
            plsc.store_scatter(table, [lc], m, mask=f)
            g3 = plsc.load_gather(table, [lc], mask=f)
            return f & (g3 < m)

          lax.while_loop(wcond, wbody, fail)
          return 0
        lax.fori_loop((nv // UNROLL_A) * UNROLL_A, nv, vec_tail, 0)
        return 0
      lax.fori_loop(0, nch, chunk_body, 0)

    scan_region(rb0, n0)
    scan_region(rb1, n1)

    tb = (cid * NSUB + sid) * WIN
    pltpu.sync_copy(table, t_hbm.at[pl.ds(tb, WIN)])
    plsc.subcore_barrier()

    mb = sid * MR
    pltpu.sync_copy(maxx_hbm.at[pl.ds(wbase + mb, MR)], mbuf)
    for half in range(NSUB // MERGE_B):
      descs = []
      for jj in range(MERGE_B):
        j = half * MERGE_B + jj
        jb = (cid * NSUB + j) * WIN
        descs.append(
            pltpu.async_copy(t_hbm.at[pl.ds(jb + mb, MR)],
                             tbuf.at[jj], sem_a))
      for d in descs:
        d.wait()

      def merge_body(i, _):
        sl = pl.ds(i * LANES, LANES)
        acc = mbuf[sl]
        for jj in range(MERGE_B):
          acc = jnp.maximum(acc, tbuf[jj, sl])
        mbuf[sl] = acc
        return 0
      lax.fori_loop(0, MR // LANES, merge_body, 0, unroll=2)
    pltpu.sync_copy(mbuf, m_hbm.at[pl.ds(wbase + mb, MR)])
    plsc.subcore_barrier()
    return 0
  lax.fori_loop(0, NWIN // NCORE, pass_body, 0)


MR = WIN // NSUB


def _normalize_body(values_hbm, col2_hbm, m_hbm, bias_hbm, out_hbm,
                    cb, vb, gb, bb, ob, sem):
  cid = lax.axis_index("c")
  sid = lax.axis_index("s")
  wid = sid * NCORE + cid

  def chunk_body(ch, _):
    e0 = wid * EP_B + ch * CH_B
    r0 = wid * (EP_B // 128) + ch * ROWS_B
    pltpu.sync_copy(col2_hbm.at[pl.ds(r0, ROWS_B)], cb)
    pltpu.sync_copy(values_hbm.at[pl.ds(e0, CH_B)], vb)

    descs = []
    for j in range(ROWS_B):
      descs.append(
          pltpu.async_copy(m_hbm.at[cb.at[j]],
                           gb.at[pl.ds(j * 128, 128)], sem))
      descs.append(
          pltpu.async_copy(bias_hbm.at[cb.at[j]],
                           bb.at[pl.ds(j * 128, 128)], sem))
    for d in descs:
      d.wait()

    def vec_body(k, _):
      sl = pl.ds(k * LANES, LANES)
      g = gb[sl]
      v = vb[sl]
      b = bb[sl]
      q = v / jnp.maximum(g, EPS_K)
      q = jnp.minimum(jnp.maximum(q, -1.0), 1.0)
      ob[sl] = q + b
      return 0
    lax.fori_loop(0, CH_B // LANES, vec_body, 0, unroll=4)

    pltpu.sync_copy(ob, out_hbm.at[pl.ds(e0, CH_B)])
    return 0
  lax.fori_loop(0, NCH_B, chunk_body, 0)


def _make_kernels():
  mesh = plsc.VectorSubcoreMesh(core_axis_name="c", subcore_axis_name="s")
  params = pltpu.CompilerParams(needs_layout_passes=False)
  kernel_a0 = pl.kernel(
      _bin_body,
      out_type=[
          jax.ShapeDtypeStruct((NW * 2 * REG,), jnp.int32),
          jax.ShapeDtypeStruct((NW * 2 * REG,), jnp.float32),
          jax.ShapeDtypeStruct((2, NW, LANES), jnp.int32),
      ],
      mesh=mesh,
      compiler_params=params,
      scratch_types=[
          pltpu.VMEM((CH0,), jnp.int32),
          pltpu.VMEM((CH0,), jnp.float32),
          pltpu.VMEM((CH0 + LANES,), jnp.int32),
          pltpu.VMEM((CH0 + LANES,), jnp.float32),
          pltpu.VMEM((CH0 + LANES,), jnp.int32),
          pltpu.VMEM((CH0 + LANES,), jnp.float32),
          pltpu.VMEM((LANES,), jnp.int32),
      ],
  )
  kernel_a1 = pl.kernel(
      _scatter_max_body,
      out_type=[
          jax.ShapeDtypeStruct((COLS_PAD,), jnp.float32),
          jax.ShapeDtypeStruct((NCORE * NSUB * WIN,), jnp.float32),
      ],
      mesh=mesh,
      compiler_params=params,
      scratch_types=[
          pltpu.VMEM((WIN,), jnp.float32),
          pltpu.VMEM((CH_A,), jnp.int32),
          pltpu.VMEM((CH_A,), jnp.float32),
          pltpu.VMEM((MR,), jnp.float32),
          pltpu.VMEM((MERGE_B, MR), jnp.float32),
          pltpu.VMEM((LANES,), jnp.int32),
          pltpu.SemaphoreType.DMA,
      ],
  )
  kernel_b = pl.kernel(
      _normalize_body,
      out_type=jax.ShapeDtypeStruct((NNZ_K,), jnp.float32),
      mesh=mesh,
      compiler_params=params,
      scratch_types=[
          pltpu.VMEM((ROWS_B, 128), jnp.int32),
          pltpu.VMEM((CH_B,), jnp.float32),
          pltpu.VMEM((CH_B,), jnp.float32),
          pltpu.VMEM((CH_B,), jnp.float32),
          pltpu.VMEM((CH_B,), jnp.float32),
          pltpu.SemaphoreType.DMA,
      ],
  )
  return kernel_a0, kernel_a1, kernel_b


_KERNEL_A0, _KERNEL_A1, _KERNEL_B = _make_kernels()


@jax.jit
def kernel(values, col_idx, max_x, bias_x):
  col = col_idx.astype(jnp.int32)
  maxx_p = jnp.pad(max_x, (0, COLS_PAD - N_COLS))
  bias_p = jnp.pad(bias_x, (0, COLS_PAD - N_COLS))
  col2 = col.reshape(NNZ_K // 128, 128)
  c_scr, v_scr, cnts = _KERNEL_A0(values, col)
  m, _ = _KERNEL_A1(c_scr, v_scr, cnts, maxx_p)
  out = _KERNEL_B(values, col2, m, bias_p)
  return out


# speedup vs baseline: 5.0463x; 1.0120x over previous
import jax
import jax.numpy as jnp
from jax import lax
from jax.experimental import pallas as pl
from jax.experimental.pallas import tpu as pltpu
from jax.experimental.pallas import tpu_sc as plsc

NNZ_K = 1638400
N_COLS = 1000000
COLS_PAD = 1 << 20
WIN = 1 << 16
NWIN = COLS_PAD // WIN
HALF_BIT = 1 << 19
LANES = 16
EPS_K = 1e-5

NSUB = 16
NCORE = 2
NW = NCORE * NSUB

SLICE0 = NNZ_K // NW
CH0 = 2048
NCH0 = SLICE0 // CH0
REG = SLICE0 + 64

CH_A = 8192
UNROLL_A = 8
MERGE_B = 8

EP_B = NNZ_K // NW
CH_B = 10240
NCH_B = EP_B // CH_B
ROWS_B = CH_B // 128


def _popcnt(mask):
  return plsc.all_reduce_population_count(mask)[0]


def _bin_body(values_hbm, col_hbm, c_scr, v_scr, cnt_hbm,
              inc, inv, oc0, ov0, oc1, ov1, cntv):
  cid = lax.axis_index("c")
  sid = lax.axis_index("s")
  wid = sid * NCORE + cid
  base0 = (wid * 2 + 0) * REG
  base1 = (wid * 2 + 1) * REG
  sent0 = jnp.zeros((LANES,), jnp.int32)
  sent1 = jnp.full((LANES,), HALF_BIT, jnp.int32)
  zf = jnp.zeros((LANES,), jnp.float32)

  def chunk_body(ch, hoffs):
    hoff0, hoff1 = hoffs
    e0 = wid * SLICE0 + ch * CH0
    pltpu.sync_copy(col_hbm.at[pl.ds(e0, CH0)], inc)
    pltpu.sync_copy(values_hbm.at[pl.ds(e0, CH0)], inv)

    def vec_body(k, offs):
      off0, off1 = offs
      sl = pl.ds(k * LANES, LANES)
      c = inc[sl]
      v = inv[sl]
      m1 = c >= HALF_BIT
      m0 = jnp.logical_not(m1)
      plsc.store_compressed(oc0.at[pl.ds(off0, LANES)], c, mask=m0)
      plsc.store_compressed(ov0.at[pl.ds(off0, LANES)], v, mask=m0)
      plsc.store_compressed(oc1.at[pl.ds(off1, LANES)], c, mask=m1)
      plsc.store_compressed(ov1.at[pl.ds(off1, LANES)], v, mask=m1)
      n0 = _popcnt(m0)
      return (off0 + n0, off1 + (LANES - n0))
    off0, off1 = lax.fori_loop(0, CH0 // LANES, vec_body, (0, 0), unroll=2)

    oc0[pl.ds(off0, LANES)] = sent0
    ov0[pl.ds(off0, LANES)] = zf
    oc1[pl.ds(off1, LANES)] = sent1
    ov1[pl.ds(off1, LANES)] = zf
    off0p = (off0 + LANES - 1) & ~(LANES - 1)
    off1p = (off1 + LANES - 1) & ~(LANES - 1)
    d0 = pl.multiple_of(base0 + hoff0, LANES)
    d1 = pl.multiple_of(base1 + hoff1, LANES)
    pltpu.sync_copy(oc0, c_scr.at[pl.ds(d0, CH0 + LANES)])
    pltpu.sync_copy(ov0, v_scr.at[pl.ds(d0, CH0 + LANES)])
    pltpu.sync_copy(oc1, c_scr.at[pl.ds(d1, CH0 + LANES)])
    pltpu.sync_copy(ov1, v_scr.at[pl.ds(d1, CH0 + LANES)])
    return (hoff0 + off0p, hoff1 + off1p)

  hoff0, hoff1 = lax.fori_loop(0, NCH0, chunk_body, (0, 0))
  cntv[pl.ds(0, LANES)] = jnp.full((LANES,), hoff0, jnp.int32)
  pltpu.sync_copy(cntv, cnt_hbm.at[0, wid])
  cntv[pl.ds(0, LANES)] = jnp.full((LANES,), hoff1, jnp.int32)
  pltpu.sync_copy(cntv, cnt_hbm.at[1, wid])


def _scatter_max_body(c_scr, v_scr, cnt_hbm, maxx_hbm, m_hbm, t_hbm,
                      table, cbuf, vbuf, mbuf, tbuf, cntv, sem_a):
  cid = lax.axis_index("c")
  sid = lax.axis_index("s")
  h = cid
  p0 = 2 * sid
  p1 = 2 * sid + 1

  pltpu.sync_copy(cnt_hbm.at[h, p0], cntv)
  n0 = cntv[pl.ds(0, LANES)][0]
  pltpu.sync_copy(cnt_hbm.at[h, p1], cntv)
  n1 = cntv[pl.ds(0, LANES)][0]
  rb0 = (p0 * 2 + h) * REG
  rb1 = (p1 * 2 + h) * REG

  def pass_body(p, _):
    win_id = cid * (NWIN // NCORE) + p
    wbase = win_id * WIN

    def zero_body(i, _):
      table[pl.ds(i * LANES, LANES)] = jnp.zeros((LANES,), jnp.float32)
      return 0
    lax.fori_loop(0, WIN // LANES, zero_body, 0, unroll=8)

    def scan_region(rbase, n):
      nch = (n + CH_A - 1) >> 13

      def chunk_body(ch, _):
        e0 = pl.multiple_of(rbase + ch * CH_A, LANES)
        pltpu.sync_copy(c_scr.at[pl.ds(e0, CH_A)], cbuf)
        pltpu.sync_copy(v_scr.at[pl.ds(e0, CH_A)], vbuf)
        nv = jnp.minimum(CH_A, n - ch * CH_A) >> 4


        def vec_body(k4, _):
          lcs, inws, avs = [], [], []
          for u in range(UNROLL_A):
            sl = pl.ds((k4 * UNROLL_A + u) * LANES, LANES)
            c = cbuf[sl]
            v = vbuf[sl]
            inws.append((c >> 16) == win_id)
            lcs.append(c & (WIN - 1))
            avs.append(jnp.abs(v))
          gs = [plsc.load_gather(table, [lcs[u]], mask=inws[u])
                for u in range(UNROLL_A)]
          ms = [jnp.maximum(gs[u], avs[u]) for u in range(UNROLL_A)]
          for u in range(UNROLL_A):
            plsc.store_scatter(table, [lcs[u]], ms[u], mask=inws[u])
          fails = []
          for u in range(UNROLL_A):
            g2 = plsc.load_gather(table, [lcs[u]], mask=inws[u])
            fails.append(inws[u] & (g2 < ms[u]))
          anyf = fails[0]
          for u in range(1, UNROLL_A):
            anyf = anyf | fails[u]

          @pl.when(jnp.any(anyf))
          def _fix():
            for u in range(UNROLL_A):
              lc, m = lcs[u], ms[u]

              def wcond(st):
                return jnp.any(st)

              def wbody(f):
                g3 = plsc.load_gather(table, [lc], mask=f)
                plsc.store_scatter(table, [lc], jnp.maximum(g3, m), mask=f)
                g4 = plsc.load_gather(table, [lc], mask=f)
                return f & (g4 < m)

              lax.while_loop(wcond, wbody, fails[u])
          return 0
        lax.fori_loop(0, nv // UNROLL_A, vec_body, 0)

        def vec_tail(k, _):
          sl = pl.ds(k * LANES, LANES)
          c = cbuf[sl]
          v = vbuf[sl]
          inw = (c >> 16) == win_id
          lc = c & (WIN - 1)
          a = jnp.abs(v)
          g = plsc.load_gather(table, [lc], mask=inw)
          m = jnp.maximum(g, a)
          plsc.store_scatter(table, [lc], m, mask=inw)
          g2 = plsc.load_gather(table, [lc], mask=inw)
          fail = inw & (g2 < m)

          def wcond(st):
            return jnp.any(st)

          def wbody(f):
            plsc.store_scatter(table, [lc], m, mask=f)
            g3 = plsc.load_gather(table, [lc], mask=f)
            return f & (g3 < m)

          lax.while_loop(wcond, wbody, fail)
          return 0
        lax.fori_loop((nv // UNROLL_A) * UNROLL_A, nv, vec_tail, 0)
        return 0
      lax.fori_loop(0, nch, chunk_body, 0)

    scan_region(rb0, n0)
    scan_region(rb1, n1)

    tb = (cid * NSUB + sid) * WIN
    pltpu.sync_copy(table, t_hbm.at[pl.ds(tb, WIN)])
    plsc.subcore_barrier()

    mb = sid * MR
    pltpu.sync_copy(maxx_hbm.at[pl.ds(wbase + mb, MR)], mbuf)
    for half in range(NSUB // MERGE_B):
      descs = []
      for jj in range(MERGE_B):
        j = half * MERGE_B + jj
        jb = (cid * NSUB + j) * WIN
        descs.append(
            pltpu.async_copy(t_hbm.at[pl.ds(jb + mb, MR)],
                             tbuf.at[jj], sem_a))
      for d in descs:
        d.wait()

      def merge_body(i, _):
        sl = pl.ds(i * LANES, LANES)
        acc = mbuf[sl]
        for jj in range(MERGE_B):
          acc = jnp.maximum(acc, tbuf[jj, sl])
        mbuf[sl] = acc
        return 0
      lax.fori_loop(0, MR // LANES, merge_body, 0, unroll=2)
    pltpu.sync_copy(mbuf, m_hbm.at[pl.ds(wbase + mb, MR)])
    plsc.subcore_barrier()
    return 0
  lax.fori_loop(0, NWIN // NCORE, pass_body, 0)


MR = WIN // NSUB


def _normalize_body(values_hbm, col2_hbm, m_hbm, bias_hbm, out_hbm,
                    cb, vb, gb, bb, ob, sem):
  cid = lax.axis_index("c")
  sid = lax.axis_index("s")
  wid = sid * NCORE + cid

  def chunk_body(ch, _):
    e0 = wid * EP_B + ch * CH_B
    r0 = wid * (EP_B // 128) + ch * ROWS_B
    pltpu.sync_copy(col2_hbm.at[pl.ds(r0, ROWS_B)], cb)
    pltpu.sync_copy(values_hbm.at[pl.ds(e0, CH_B)], vb)

    descs = []
    for j in range(ROWS_B):
      descs.append(
          pltpu.async_copy(m_hbm.at[cb.at[j]],
                           gb.at[pl.ds(j * 128, 128)], sem))
      descs.append(
          pltpu.async_copy(bias_hbm.at[cb.at[j]],
                           bb.at[pl.ds(j * 128, 128)], sem))
    for d in descs:
      d.wait()

    def vec_body(k, _):
      sl = pl.ds(k * LANES, LANES)
      g = gb[sl]
      v = vb[sl]
      b = bb[sl]
      q = v / jnp.maximum(g, EPS_K)
      q = jnp.minimum(jnp.maximum(q, -1.0), 1.0)
      ob[sl] = q + b
      return 0
    lax.fori_loop(0, CH_B // LANES, vec_body, 0, unroll=4)

    pltpu.sync_copy(ob, out_hbm.at[pl.ds(e0, CH_B)])
    return 0
  lax.fori_loop(0, NCH_B, chunk_body, 0)


def _make_kernels():
  mesh = plsc.VectorSubcoreMesh(core_axis_name="c", subcore_axis_name="s")
  params = pltpu.CompilerParams(needs_layout_passes=False)
  kernel_a0 = pl.kernel(
      _bin_body,
      out_type=[
          jax.ShapeDtypeStruct((NW * 2 * REG,), jnp.int32),
          jax.ShapeDtypeStruct((NW * 2 * REG,), jnp.float32),
          jax.ShapeDtypeStruct((2, NW, LANES), jnp.int32),
      ],
      mesh=mesh,
      compiler_params=params,
      scratch_types=[
          pltpu.VMEM((CH0,), jnp.int32),
          pltpu.VMEM((CH0,), jnp.float32),
          pltpu.VMEM((CH0 + LANES,), jnp.int32),
          pltpu.VMEM((CH0 + LANES,), jnp.float32),
          pltpu.VMEM((CH0 + LANES,), jnp.int32),
          pltpu.VMEM((CH0 + LANES,), jnp.float32),
          pltpu.VMEM((LANES,), jnp.int32),
      ],
  )
  kernel_a1 = pl.kernel(
      _scatter_max_body,
      out_type=[
          jax.ShapeDtypeStruct((COLS_PAD,), jnp.float32),
          jax.ShapeDtypeStruct((NCORE * NSUB * WIN,), jnp.float32),
      ],
      mesh=mesh,
      compiler_params=params,
      scratch_types=[
          pltpu.VMEM((WIN,), jnp.float32),
          pltpu.VMEM((CH_A,), jnp.int32),
          pltpu.VMEM((CH_A,), jnp.float32),
          pltpu.VMEM((MR,), jnp.float32),
          pltpu.VMEM((MERGE_B, MR), jnp.float32),
          pltpu.VMEM((LANES,), jnp.int32),
          pltpu.SemaphoreType.DMA,
      ],
  )
  kernel_b = pl.kernel(
      _normalize_body,
      out_type=jax.ShapeDtypeStruct((NNZ_K,), jnp.float32),
      mesh=mesh,
      compiler_params=params,
      scratch_types=[
          pltpu.VMEM((ROWS_B, 128), jnp.int32),
          pltpu.VMEM((CH_B,), jnp.float32),
          pltpu.VMEM((CH_B,), jnp.float32),
          pltpu.VMEM((CH_B,), jnp.float32),
          pltpu.VMEM((CH_B,), jnp.float32),
          pltpu.SemaphoreType.DMA,
      ],
  )
  return kernel_a0, kernel_a1, kernel_b


_KERNEL_A0, _KERNEL_A1, _KERNEL_B = _make_kernels()


@jax.jit
def kernel(values, col_idx, max_x, bias_x):
  col = col_idx.astype(jnp.int32)
  maxx_p = jnp.pad(max_x, (0, COLS_PAD - N_COLS))
  bias_p = jnp.pad(bias_x, (0, COLS_PAD - N_COLS))
  col2 = col.reshape(NNZ_K // 128, 128)
  c_scr, v_scr, cnts = _KERNEL_A0(values, col)
  m, _ = _KERNEL_A1(c_scr, v_scr, cnts, maxx_p)
  out = _KERNEL_B(values, col2, m, bias_p)
  return out
